# baseline (speedup 1.0000x reference)
import functools

import jax
import jax.numpy as jnp
from jax.experimental import pallas as pl
from jax.experimental.pallas import tpu as pltpu

_BLK = 1000


def _dg(a, b, ca, cb):
    return jax.lax.dot_general(
        a, b, dimension_numbers=(((ca,), (cb,)), ((), ())),
        preferred_element_type=jnp.float32)


def _lstm_kernel(h_dim, x_ref, h_ref, c_ref, wx_ref, wh_ref, b_ref,
                 wci_ref, wcf_ref, wco_ref, fcw_ref, fcb_ref, eye_ref,
                 out_ref, hn_ref, cn_ref):
    x = x_ref[...]
    h = h_ref[...]
    c = c_ref[...]
    eye = eye_ref[...]

    pre = _dg(wx_ref[...], x, 0, 1)
    pre = pre + _dg(wh_ref[...], h, 0, 1)
    pre = pre + b_ref[...]
    ct = _dg(eye, c, 1, 1)
    i_g = jax.nn.sigmoid(pre[0 * h_dim:1 * h_dim, :] + wci_ref[...] * ct)
    f_g = jax.nn.sigmoid(pre[1 * h_dim:2 * h_dim, :] + wcf_ref[...] * ct)
    t_g = jnp.tanh(pre[2 * h_dim:3 * h_dim, :])
    cn_t = f_g * ct + i_g * t_g
    o_g = jax.nn.sigmoid(pre[3 * h_dim:4 * h_dim, :] + wco_ref[...] * cn_t)
    hn_t = o_g * jnp.tanh(cn_t)
    cn_ref[...] = _dg(cn_t, eye, 0, 0)
    hn_ref[...] = _dg(hn_t, eye, 0, 0)
    relu_h = jnp.maximum(hn_t, 0.0)
    out_ref[...] = _dg(relu_h, fcw_ref[...], 0, 0) + fcb_ref[...]


def kernel(x, edge_index, edge_weight, h, c,
           W_xi, b_xi, W_hi, b_hi, W_xf, b_xf, W_hf, b_hf,
           W_xc, b_xc, W_hc, b_hc, W_xo, b_xo, W_ho, b_ho,
           w_ci, w_cf, w_co, b_i, b_f, b_c, b_o, fc_w, fc_b):
    del edge_index, edge_weight
    f_in = x.shape[1]
    h_dim = h.shape[1]
    wx = jnp.concatenate([W_xi, W_xf, W_xc, W_xo], axis=1)
    wh = jnp.concatenate([W_hi, W_hf, W_hc, W_ho], axis=1)
    bias = jnp.concatenate([b_xi + b_hi + b_i[0],
                            b_xf + b_hf + b_f[0],
                            b_xc + b_hc + b_c[0],
                            b_xo + b_ho + b_o[0]])[:, None]
    wci_c = w_ci.T
    wcf_c = w_cf.T
    wco_c = w_co.T
    fcb = fc_b.reshape(1, 1)
    eye = jnp.eye(h_dim, dtype=jnp.float32)

    n = x.shape[0]
    grid = (n // _BLK,)
    row = lambda i: (i, 0)
    full = lambda i: (0, 0)
    out, h_new, c_new = pl.pallas_call(
        functools.partial(_lstm_kernel, h_dim),
        grid=grid,
        in_specs=[
            pl.BlockSpec((_BLK, f_in), row),
            pl.BlockSpec((_BLK, h_dim), row),
            pl.BlockSpec((_BLK, h_dim), row),
            pl.BlockSpec((f_in, 4 * h_dim), full),
            pl.BlockSpec((h_dim, 4 * h_dim), full),
            pl.BlockSpec((4 * h_dim, 1), full),
            pl.BlockSpec((h_dim, 1), full),
            pl.BlockSpec((h_dim, 1), full),
            pl.BlockSpec((h_dim, 1), full),
            pl.BlockSpec((h_dim, 1), full),
            pl.BlockSpec((1, 1), full),
            pl.BlockSpec((h_dim, h_dim), full),
        ],
        out_specs=[
            pl.BlockSpec((_BLK, 1), row),
            pl.BlockSpec((_BLK, h_dim), row),
            pl.BlockSpec((_BLK, h_dim), row),
        ],
        out_shape=[
            jax.ShapeDtypeStruct((n, 1), jnp.float32),
            jax.ShapeDtypeStruct((n, h_dim), jnp.float32),
            jax.ShapeDtypeStruct((n, h_dim), jnp.float32),
        ],
        compiler_params=pltpu.CompilerParams(
            dimension_semantics=("arbitrary",),
        ),
    )(x, h, c, wx, wh, bias, wci_c, wcf_c, wco_c, fc_w, fcb, eye)
    return (out, h_new, c_new)


# speedup vs baseline: 1.0510x; 1.0510x over previous
import functools

import jax
import jax.numpy as jnp
from jax.experimental import pallas as pl
from jax.experimental.pallas import tpu as pltpu

_BLK = 2000


def _dg(a, b, ca, cb):
    return jax.lax.dot_general(
        a, b, dimension_numbers=(((ca,), (cb,)), ((), ())),
        preferred_element_type=jnp.float32)


def _lstm_kernel(h_dim, x_ref, h_ref, c_ref, wx_ref, wh_ref, b_ref,
                 wci_ref, wcf_ref, wco_ref, fcw_ref, fcb_ref, eye_ref,
                 out_ref, hn_ref, cn_ref):
    x = x_ref[...]
    h = h_ref[...]
    c = c_ref[...]
    eye = eye_ref[...]

    pre = _dg(wx_ref[...], x, 0, 1)
    pre = pre + _dg(wh_ref[...], h, 0, 1)
    pre = pre + b_ref[...]
    ct = _dg(eye, c, 1, 1)
    i_g = jax.nn.sigmoid(pre[0 * h_dim:1 * h_dim, :] + wci_ref[...] * ct)
    f_g = jax.nn.sigmoid(pre[1 * h_dim:2 * h_dim, :] + wcf_ref[...] * ct)
    t_g = jnp.tanh(pre[2 * h_dim:3 * h_dim, :])
    cn_t = f_g * ct + i_g * t_g
    o_g = jax.nn.sigmoid(pre[3 * h_dim:4 * h_dim, :] + wco_ref[...] * cn_t)
    hn_t = o_g * jnp.tanh(cn_t)
    cn_ref[...] = _dg(cn_t, eye, 0, 0)
    hn_ref[...] = _dg(hn_t, eye, 0, 0)
    relu_h = jnp.maximum(hn_t, 0.0)
    out_ref[...] = _dg(relu_h, fcw_ref[...], 0, 0) + fcb_ref[...]


def kernel(x, edge_index, edge_weight, h, c,
           W_xi, b_xi, W_hi, b_hi, W_xf, b_xf, W_hf, b_hf,
           W_xc, b_xc, W_hc, b_hc, W_xo, b_xo, W_ho, b_ho,
           w_ci, w_cf, w_co, b_i, b_f, b_c, b_o, fc_w, fc_b):
    del edge_index, edge_weight
    f_in = x.shape[1]
    h_dim = h.shape[1]
    wx = jnp.concatenate([W_xi, W_xf, W_xc, W_xo], axis=1)
    wh = jnp.concatenate([W_hi, W_hf, W_hc, W_ho], axis=1)
    bias = jnp.concatenate([b_xi + b_hi + b_i[0],
                            b_xf + b_hf + b_f[0],
                            b_xc + b_hc + b_c[0],
                            b_xo + b_ho + b_o[0]])[:, None]
    wci_c = w_ci.T
    wcf_c = w_cf.T
    wco_c = w_co.T
    fcb = fc_b.reshape(1, 1)
    eye = jnp.eye(h_dim, dtype=jnp.float32)

    n = x.shape[0]
    grid = (n // _BLK,)
    row = lambda i: (i, 0)
    full = lambda i: (0, 0)
    out, h_new, c_new = pl.pallas_call(
        functools.partial(_lstm_kernel, h_dim),
        grid=grid,
        in_specs=[
            pl.BlockSpec((_BLK, f_in), row),
            pl.BlockSpec((_BLK, h_dim), row),
            pl.BlockSpec((_BLK, h_dim), row),
            pl.BlockSpec((f_in, 4 * h_dim), full),
            pl.BlockSpec((h_dim, 4 * h_dim), full),
            pl.BlockSpec((4 * h_dim, 1), full),
            pl.BlockSpec((h_dim, 1), full),
            pl.BlockSpec((h_dim, 1), full),
            pl.BlockSpec((h_dim, 1), full),
            pl.BlockSpec((h_dim, 1), full),
            pl.BlockSpec((1, 1), full),
            pl.BlockSpec((h_dim, h_dim), full),
        ],
        out_specs=[
            pl.BlockSpec((_BLK, 1), row),
            pl.BlockSpec((_BLK, h_dim), row),
            pl.BlockSpec((_BLK, h_dim), row),
        ],
        out_shape=[
            jax.ShapeDtypeStruct((n, 1), jnp.float32),
            jax.ShapeDtypeStruct((n, h_dim), jnp.float32),
            jax.ShapeDtypeStruct((n, h_dim), jnp.float32),
        ],
        compiler_params=pltpu.CompilerParams(
            dimension_semantics=("arbitrary",),
        ),
    )(x, h, c, wx, wh, bias, wci_c, wcf_c, wco_c, fc_w, fcb, eye)
    return (out, h_new, c_new)
